# baseline (speedup 1.0000x reference)
import functools

import jax
import jax.numpy as jnp
from jax import lax
from jax.experimental import pallas as pl
from jax.experimental.pallas import tpu as pltpu
from jax.experimental.pallas import tpu_sc as plsc

N = 10000
D = 128
E = 320000
NC = 2
NS = 16
NW = NC * NS
EW = E // NW
CH = 100
NCHUNK = EW // CH
NHALF = 4
HALF = NCHUNK // NHALF
NBUF = 3
STRIPE = 624
TAIL = N - NS * STRIPE
PAR_PAD = 10240
PW = PAR_PAD // NW
PCH = 64
PSTEPS = PW // PCH


def _sc_sparse(items, operations, isrc, idst, osrc, odst, parents, zrows):
    mesh = plsc.VectorSubcoreMesh(
        core_axis_name="c", subcore_axis_name="s",
        num_cores=NC, num_subcores=NS)

    @functools.partial(
        pl.kernel,
        out_type=(
            jax.ShapeDtypeStruct((NC, N, D), jnp.float32),
            jax.ShapeDtypeStruct((NC, N, D), jnp.float32),
            jax.ShapeDtypeStruct((PAR_PAD, D), jnp.float32),
        ),
        mesh=mesh,
        scratch_types=[
            pltpu.MemorySpace.VMEM_SHARED((N, D), jnp.float32),
            pltpu.VMEM((HALF, CH), jnp.int32),
            pltpu.VMEM((HALF, CH), jnp.int32),
            pltpu.VMEM((CH, D), jnp.float32),
            pltpu.VMEM((CH, D), jnp.float32),
            pltpu.VMEM((CH, D), jnp.float32),
            pltpu.VMEM((PW,), jnp.int32),
            pltpu.SemaphoreType.DMA,
            pltpu.SemaphoreType.DMA,
            pltpu.SemaphoreType.DMA,
            pltpu.SemaphoreType.DMA,
            pltpu.SemaphoreType.DMA,
            pltpu.SemaphoreType.DMA,
        ],
    )
    def k(items_h, ops_h, isrc_h, idst_h, osrc_h, odst_h, par_h, zrows_h,
          child_o, opsagg_o, par_o, acc, sidx, didx, rows, rows1, rows2,
          pidx, sem, sem1, sem2, ssem, ssem1, ssem2):
        c = lax.axis_index("c")
        s = lax.axis_index("s")
        wid = s * NC + c

        def stripe_copy(src_fn, dst_fn):
            pltpu.sync_copy(src_fn(0, STRIPE), dst_fn(0, STRIPE))

            @pl.when(s == NS - 1)
            def _():
                pltpu.sync_copy(src_fn(STRIPE, TAIL), dst_fn(STRIPE, TAIL))

        def segsum(src_h, dst_h, table_h, out_h):
            base = s * STRIPE
            stripe_copy(lambda o, n: zrows_h.at[pl.ds(o, n)],
                        lambda o, n: acc.at[pl.ds(base + o, n)])
            plsc.subcore_barrier()

            bufs = (rows, rows1, rows2)
            sems = (sem, sem1, sem2)
            ssems = (ssem, ssem1, ssem2)

            def wait_gather(j, b):
                pltpu.make_async_copy(table_h.at[sidx.at[j]], bufs[b],
                                      sems[b]).wait()

            def wait_scatter(j, b):
                pltpu.make_async_copy(bufs[b], acc.at[didx.at[j]],
                                      ssems[b]).wait()

            for g in range(NHALF):
                pltpu.sync_copy(src_h.at[wid, g], sidx)
                pltpu.sync_copy(dst_h.at[wid, g], didx)

                for j in range(NBUF):
                    pltpu.async_copy(table_h.at[sidx.at[j]], bufs[j],
                                     sems[j])

                def triple(i, _):
                    j0 = i * NBUF
                    for b in range(NBUF):
                        j = j0 + b
                        wait_gather(j, b)
                        pltpu.async_copy(bufs[b], acc.at[didx.at[j]],
                                         ssems[b], add=True)
                        bp = (b - 1) % NBUF

                        def retire(jp=j - 1, bb=bp):
                            wait_scatter(jp, bb)

                            @pl.when(jp + NBUF < HALF)
                            def _():
                                pltpu.async_copy(
                                    table_h.at[sidx.at[jp + NBUF]],
                                    bufs[bb], sems[bb])

                        if b == 0:
                            @pl.when(i > 0)
                            def _():
                                retire()
                        else:
                            retire()
                    return ()

                lax.fori_loop(0, HALF // NBUF, triple, ())
                jt = HALF - 1
                bt = jt % NBUF
                wait_gather(jt, bt)
                pltpu.async_copy(bufs[bt], acc.at[didx.at[jt]], ssems[bt],
                                 add=True)
                wait_scatter(jt - 1, (bt - 1) % NBUF)
                wait_scatter(jt, bt)
            plsc.subcore_barrier()
            stripe_copy(lambda o, n: acc.at[pl.ds(base + o, n)],
                        lambda o, n: out_h.at[c, pl.ds(base + o, n)])
            plsc.subcore_barrier()

        segsum(isrc_h, idst_h, items_h, child_o)
        segsum(osrc_h, odst_h, ops_h, opsagg_o)

        pltpu.sync_copy(par_h.at[pl.ds(wid * PW, PW)], pidx)
        for t in range(PSTEPS):
            pltpu.async_copy(items_h.at[pidx.at[pl.ds(t * PCH, PCH)]],
                             rows.at[pl.ds(0, PCH)], sem).wait()
            pltpu.sync_copy(rows.at[pl.ds(0, PCH)],
                            par_o.at[pl.ds(wid * PW + t * PCH, PCH)])

    return k(items, operations, isrc, idst, osrc, odst, parents, zrows)


def _elu(x):
    return jnp.where(x > 0, x, jnp.exp(x) - 1.0)


def _mlp(x, w1, b1, w2, b2, w3, b3):
    x = _elu(jnp.dot(x, w1, preferred_element_type=jnp.float32) + b1)
    x = _elu(jnp.dot(x, w2, preferred_element_type=jnp.float32) + b2)
    return jnp.dot(x, w3, preferred_element_type=jnp.float32) + b3


R = 1000


def _tc_mlps(items, par_rows, child_part, ops_part, wlist):
    grid = (N // R,)

    def body(items_b, par_b, ch_b, op_b,
             ws1, bs1, ws2, bs2, ws3, bs3,
             wp1, bp1, wp2, bp2, wp3, bp3,
             wc1, bc1, wc2, bc2, wc3, bc3,
             wo1, bo1, wo2, bo2, wo3, bo3,
             wm1, bm1, wm2, bm2, wm3, bm3,
             out_b):
        es = _mlp(items_b[...], ws1[...], bs1[...], ws2[...], bs2[...],
                  ws3[...], bs3[...])
        ep = _mlp(par_b[...], wp1[...], bp1[...], wp2[...], bp2[...],
                  wp3[...], bp3[...])
        ec = _mlp(ch_b[0] + ch_b[1], wc1[...], bc1[...], wc2[...], bc2[...],
                  wc3[...], bc3[...])
        eo = _mlp(op_b[0] + op_b[1], wo1[...], bo1[...], wo2[...], bo2[...],
                  wo3[...], bo3[...])
        x = jnp.concatenate([ep, ec, eo, es], axis=1)
        out = _mlp(x, wm1[...], bm1[...], wm2[...], bm2[...], wm3[...],
                   bm3[...])
        rid = pl.program_id(0) * R + lax.broadcasted_iota(jnp.int32, (R, 1), 0)
        out_b[...] = jnp.where(rid == N - 1, 0.0, out)

    row_spec = pl.BlockSpec((R, D), lambda i: (i, 0))
    part_spec = pl.BlockSpec((NC, R, D), lambda i: (0, i, 0))

    def wspec(a):
        return pl.BlockSpec(a.shape, lambda i: (0,) * a.ndim)

    return pl.pallas_call(
        body,
        grid=grid,
        in_specs=[row_spec, row_spec, part_spec, part_spec]
        + [wspec(a) for a in wlist],
        out_specs=row_spec,
        out_shape=jax.ShapeDtypeStruct((N, D), jnp.float32),
        compiler_params=pltpu.CompilerParams(
            dimension_semantics=("arbitrary",)),
    )(items, par_rows, child_part, ops_part, *wlist)


def kernel(items, parents, operations, item_edge_index, op_edge_index, params):
    eshape = (NW, NHALF, HALF, CH)
    isrc = item_edge_index[1].reshape(eshape).astype(jnp.int32)
    idst = item_edge_index[0].reshape(eshape).astype(jnp.int32)
    osrc = op_edge_index[1].reshape(eshape).astype(jnp.int32)
    odst = op_edge_index[0].reshape(eshape).astype(jnp.int32)
    par_pad = jnp.concatenate(
        [parents.astype(jnp.int32),
         jnp.zeros((PAR_PAD - N,), jnp.int32)])
    zrows = jnp.zeros((STRIPE + TAIL, D), jnp.float32)

    child_part, ops_part, par_rows = _sc_sparse(
        items, operations, isrc, idst, osrc, odst, par_pad, zrows)

    wlist = []
    for name in ("mlp_self", "mlp_parent", "mlp_children", "mlp_operations",
                 "mlp_combined"):
        p = params[name]
        for l in ("l1", "l2", "l3"):
            w, b = p[l]
            wlist.append(w)
            wlist.append(b.reshape(1, -1))

    return _tc_mlps(items, par_rows[:N], child_part, ops_part, wlist)


# speedup vs baseline: 1.0433x; 1.0433x over previous
import functools

import jax
import jax.numpy as jnp
from jax import lax
from jax.experimental import pallas as pl
from jax.experimental.pallas import tpu as pltpu
from jax.experimental.pallas import tpu_sc as plsc

N = 10000
D = 128
E = 320000
NC = 2
NS = 16
NW = NC * NS
EW = E // NW
CH = 100
NCHUNK = EW // CH
NHALF = 4
HALF = NCHUNK // NHALF
NBUF = 3
STRIPE = 624
TAIL = N - NS * STRIPE
PAR_PAD = 10240
PW = PAR_PAD // NW
PCH = 64
PSTEPS = PW // PCH


def _sc_sparse(items, operations, isrc, idst, osrc, odst, parents, zrows):
    mesh = plsc.VectorSubcoreMesh(
        core_axis_name="c", subcore_axis_name="s",
        num_cores=NC, num_subcores=NS)

    @functools.partial(
        pl.kernel,
        out_type=(
            jax.ShapeDtypeStruct((NC, N, D), jnp.float32),
            jax.ShapeDtypeStruct((NC, N, D), jnp.float32),
            jax.ShapeDtypeStruct((PAR_PAD, D), jnp.float32),
        ),
        mesh=mesh,
        scratch_types=[
            pltpu.MemorySpace.VMEM_SHARED((N, D), jnp.float32),
            pltpu.VMEM((HALF, CH), jnp.int32),
            pltpu.VMEM((HALF, CH), jnp.int32),
            pltpu.VMEM((CH, D), jnp.float32),
            pltpu.VMEM((CH, D), jnp.float32),
            pltpu.VMEM((CH, D), jnp.float32),
            pltpu.VMEM((PW,), jnp.int32),
            pltpu.SemaphoreType.DMA,
            pltpu.SemaphoreType.DMA,
            pltpu.SemaphoreType.DMA,
        ],
    )
    def k(items_h, ops_h, isrc_h, idst_h, osrc_h, odst_h, par_h, zrows_h,
          child_o, opsagg_o, par_o, acc, sidx, didx, rows, rows1, rows2,
          pidx, sem, sem1, sem2):
        c = lax.axis_index("c")
        s = lax.axis_index("s")
        wid = s * NC + c

        def stripe_copy(src_fn, dst_fn):
            pltpu.sync_copy(src_fn(0, STRIPE), dst_fn(0, STRIPE))

            @pl.when(s == NS - 1)
            def _():
                pltpu.sync_copy(src_fn(STRIPE, TAIL), dst_fn(STRIPE, TAIL))

        def segsum(src_h, dst_h, table_h, out_h):
            base = s * STRIPE
            stripe_copy(lambda o, n: zrows_h.at[pl.ds(o, n)],
                        lambda o, n: acc.at[pl.ds(base + o, n)])
            plsc.subcore_barrier()

            bufs = (rows, rows1, rows2)
            sems = (sem, sem1, sem2)

            def wait_gather(j, b):
                pltpu.make_async_copy(table_h.at[sidx.at[j]], bufs[b],
                                      sems[b]).wait()

            for g in range(NHALF):
                pltpu.sync_copy(src_h.at[wid, g], sidx)
                pltpu.sync_copy(dst_h.at[wid, g], didx)

                for j in range(NBUF):
                    pltpu.async_copy(table_h.at[sidx.at[j]], bufs[j],
                                     sems[j])

                def triple(i, _):
                    j0 = i * NBUF
                    for b in range(NBUF):
                        j = j0 + b
                        wait_gather(j, b)
                        pltpu.sync_copy(bufs[b], acc.at[didx.at[j]],
                                        add=True)

                        @pl.when(j + NBUF < HALF)
                        def _(jn=j + NBUF, bb=b):
                            pltpu.async_copy(table_h.at[sidx.at[jn]],
                                             bufs[bb], sems[bb])
                    return ()

                lax.fori_loop(0, HALF // NBUF, triple, ())
                jt = HALF - 1
                bt = jt % NBUF
                wait_gather(jt, bt)
                pltpu.sync_copy(bufs[bt], acc.at[didx.at[jt]], add=True)
            plsc.subcore_barrier()
            stripe_copy(lambda o, n: acc.at[pl.ds(base + o, n)],
                        lambda o, n: out_h.at[c, pl.ds(base + o, n)])
            plsc.subcore_barrier()

        segsum(isrc_h, idst_h, items_h, child_o)
        segsum(osrc_h, odst_h, ops_h, opsagg_o)

        pltpu.sync_copy(par_h.at[pl.ds(wid * PW, PW)], pidx)
        for t in range(PSTEPS):
            pltpu.async_copy(items_h.at[pidx.at[pl.ds(t * PCH, PCH)]],
                             rows.at[pl.ds(0, PCH)], sem).wait()
            pltpu.sync_copy(rows.at[pl.ds(0, PCH)],
                            par_o.at[pl.ds(wid * PW + t * PCH, PCH)])

    return k(items, operations, isrc, idst, osrc, odst, parents, zrows)


def _elu(x):
    return jnp.where(x > 0, x, jnp.exp(x) - 1.0)


def _mlp(x, w1, b1, w2, b2, w3, b3):
    x = _elu(jnp.dot(x, w1, preferred_element_type=jnp.float32) + b1)
    x = _elu(jnp.dot(x, w2, preferred_element_type=jnp.float32) + b2)
    return jnp.dot(x, w3, preferred_element_type=jnp.float32) + b3


R = 1000


def _tc_mlps(items, par_rows, child_part, ops_part, wlist):
    grid = (N // R,)

    def body(items_b, par_b, ch_b, op_b,
             ws1, bs1, ws2, bs2, ws3, bs3,
             wp1, bp1, wp2, bp2, wp3, bp3,
             wc1, bc1, wc2, bc2, wc3, bc3,
             wo1, bo1, wo2, bo2, wo3, bo3,
             wm1, bm1, wm2, bm2, wm3, bm3,
             out_b):
        es = _mlp(items_b[...], ws1[...], bs1[...], ws2[...], bs2[...],
                  ws3[...], bs3[...])
        ep = _mlp(par_b[...], wp1[...], bp1[...], wp2[...], bp2[...],
                  wp3[...], bp3[...])
        ec = _mlp(ch_b[0] + ch_b[1], wc1[...], bc1[...], wc2[...], bc2[...],
                  wc3[...], bc3[...])
        eo = _mlp(op_b[0] + op_b[1], wo1[...], bo1[...], wo2[...], bo2[...],
                  wo3[...], bo3[...])
        x = jnp.concatenate([ep, ec, eo, es], axis=1)
        out = _mlp(x, wm1[...], bm1[...], wm2[...], bm2[...], wm3[...],
                   bm3[...])
        rid = pl.program_id(0) * R + lax.broadcasted_iota(jnp.int32, (R, 1), 0)
        out_b[...] = jnp.where(rid == N - 1, 0.0, out)

    row_spec = pl.BlockSpec((R, D), lambda i: (i, 0))
    part_spec = pl.BlockSpec((NC, R, D), lambda i: (0, i, 0))

    def wspec(a):
        return pl.BlockSpec(a.shape, lambda i: (0,) * a.ndim)

    return pl.pallas_call(
        body,
        grid=grid,
        in_specs=[row_spec, row_spec, part_spec, part_spec]
        + [wspec(a) for a in wlist],
        out_specs=row_spec,
        out_shape=jax.ShapeDtypeStruct((N, D), jnp.float32),
        compiler_params=pltpu.CompilerParams(
            dimension_semantics=("arbitrary",)),
    )(items, par_rows, child_part, ops_part, *wlist)


def kernel(items, parents, operations, item_edge_index, op_edge_index, params):
    eshape = (NW, NHALF, HALF, CH)
    isrc = item_edge_index[1].reshape(eshape).astype(jnp.int32)
    idst = item_edge_index[0].reshape(eshape).astype(jnp.int32)
    osrc = op_edge_index[1].reshape(eshape).astype(jnp.int32)
    odst = op_edge_index[0].reshape(eshape).astype(jnp.int32)
    par_pad = jnp.concatenate(
        [parents.astype(jnp.int32),
         jnp.zeros((PAR_PAD - N,), jnp.int32)])
    zrows = jnp.zeros((STRIPE + TAIL, D), jnp.float32)

    child_part, ops_part, par_rows = _sc_sparse(
        items, operations, isrc, idst, osrc, odst, par_pad, zrows)

    wlist = []
    for name in ("mlp_self", "mlp_parent", "mlp_children", "mlp_operations",
                 "mlp_combined"):
        p = params[name]
        for l in ("l1", "l2", "l3"):
            w, b = p[l]
            wlist.append(w)
            wlist.append(b.reshape(1, -1))

    return _tc_mlps(items, par_rows[:N], child_part, ops_part, wlist)


# speedup vs baseline: 1.1678x; 1.1193x over previous
import functools

import jax
import jax.numpy as jnp
from jax import lax
from jax.experimental import pallas as pl
from jax.experimental.pallas import tpu as pltpu
from jax.experimental.pallas import tpu_sc as plsc

N = 10000
D = 128
E = 320000
NC = 2
NS = 16
NW = NC * NS
EW = E // NW
CH = 100
NCHUNK = EW // CH
NHALF = 4
HALF = NCHUNK // NHALF
NBUF = 3
STRIPE = 624
TAIL = N - NS * STRIPE
PAR_PAD = 10240
PW = PAR_PAD // NW
PCH = 64
PSTEPS = PW // PCH


def _sc_parent_gather(items, parents):
    mesh = plsc.VectorSubcoreMesh(
        core_axis_name="c", subcore_axis_name="s",
        num_cores=NC, num_subcores=NS)

    @functools.partial(
        pl.kernel,
        out_type=jax.ShapeDtypeStruct((PAR_PAD, D), jnp.float32),
        mesh=mesh,
        scratch_types=[
            pltpu.VMEM((PW,), jnp.int32),
            pltpu.VMEM((PCH, D), jnp.float32),
            pltpu.SemaphoreType.DMA,
        ],
    )
    def pk(items_h, par_h, par_o, pidx, prow, sem):
        c = lax.axis_index("c")
        s = lax.axis_index("s")
        wid = s * NC + c
        pltpu.sync_copy(par_h.at[pl.ds(wid * PW, PW)], pidx)
        for t in range(PSTEPS):
            pltpu.async_copy(items_h.at[pidx.at[pl.ds(t * PCH, PCH)]],
                             prow, sem).wait()
            pltpu.sync_copy(prow,
                            par_o.at[pl.ds(wid * PW + t * PCH, PCH)])

    return pk(items, parents)


def _sc_sparse(items, operations, isrc, idst, osrc, odst, parents, zrows):
    mesh = plsc.VectorSubcoreMesh(
        core_axis_name="c", subcore_axis_name="s",
        num_cores=NC, num_subcores=NS)

    @functools.partial(
        pl.kernel,
        out_type=(
            jax.ShapeDtypeStruct((NC, N, D), jnp.float32),
            jax.ShapeDtypeStruct((NC, N, D), jnp.float32),
        ),
        mesh=mesh,
        scratch_types=[
            pltpu.MemorySpace.VMEM_SHARED((N, D), jnp.float32),
            pltpu.VMEM((HALF, CH), jnp.int32),
            pltpu.VMEM((HALF, CH), jnp.int32),
            pltpu.VMEM((CH, D), jnp.float32),
            pltpu.VMEM((CH, D), jnp.float32),
            pltpu.VMEM((CH, D), jnp.float32),
            pltpu.SemaphoreType.DMA,
            pltpu.SemaphoreType.DMA,
            pltpu.SemaphoreType.DMA,
        ],
    )
    def k(items_h, ops_h, isrc_h, idst_h, osrc_h, odst_h, zrows_h,
          child_o, opsagg_o, acc, sidx, didx, rows, rows1, rows2,
          sem, sem1, sem2):
        c = lax.axis_index("c")
        s = lax.axis_index("s")
        wid = s * NC + c

        def stripe_copy(src_fn, dst_fn):
            pltpu.sync_copy(src_fn(0, STRIPE), dst_fn(0, STRIPE))

            @pl.when(s == NS - 1)
            def _():
                pltpu.sync_copy(src_fn(STRIPE, TAIL), dst_fn(STRIPE, TAIL))

        base = s * STRIPE

        def zero_stripe():
            stripe_copy(lambda o, n: zrows_h.at[pl.ds(o, n)],
                        lambda o, n: acc.at[pl.ds(base + o, n)])

        bufs = (rows, rows1, rows2)
        sems = (sem, sem1, sem2)

        def segsum(src_h, dst_h, table_h):
            def wait_gather(j, b):
                pltpu.make_async_copy(table_h.at[sidx.at[j]], bufs[b],
                                      sems[b]).wait()

            for g in range(NHALF):
                pltpu.sync_copy(src_h.at[wid, g], sidx)
                pltpu.sync_copy(dst_h.at[wid, g], didx)

                for j in range(NBUF):
                    pltpu.async_copy(table_h.at[sidx.at[j]], bufs[j],
                                     sems[j])

                def triple(i, _):
                    j0 = i * NBUF
                    for b in range(NBUF):
                        j = j0 + b
                        wait_gather(j, b)
                        pltpu.sync_copy(bufs[b], acc.at[didx.at[j]],
                                        add=True)

                        @pl.when(j + NBUF < HALF)
                        def _(jn=j + NBUF, bb=b):
                            pltpu.async_copy(table_h.at[sidx.at[jn]],
                                             bufs[bb], sems[bb])
                    return ()

                lax.fori_loop(0, HALF // NBUF, triple, ())
                jt = HALF - 1
                bt = jt % NBUF
                wait_gather(jt, bt)
                pltpu.sync_copy(bufs[bt], acc.at[didx.at[jt]], add=True)

        zero_stripe()
        plsc.subcore_barrier()
        segsum(isrc_h, idst_h, items_h)
        plsc.subcore_barrier()
        stripe_copy(lambda o, n: acc.at[pl.ds(base + o, n)],
                    lambda o, n: child_o.at[c, pl.ds(base + o, n)])
        zero_stripe()
        plsc.subcore_barrier()

        segsum(osrc_h, odst_h, ops_h)
        plsc.subcore_barrier()
        stripe_copy(lambda o, n: acc.at[pl.ds(base + o, n)],
                    lambda o, n: opsagg_o.at[c, pl.ds(base + o, n)])

    return k(items, operations, isrc, idst, osrc, odst, zrows)


def _elu(x):
    return jnp.where(x > 0, x, jnp.exp(x) - 1.0)


def _mlp(x, w1, b1, w2, b2, w3, b3):
    x = _elu(jnp.dot(x, w1, preferred_element_type=jnp.float32) + b1)
    x = _elu(jnp.dot(x, w2, preferred_element_type=jnp.float32) + b2)
    return jnp.dot(x, w3, preferred_element_type=jnp.float32) + b3


R = 2000


_ROW_SPEC = pl.BlockSpec((R, D), lambda i: (i, 0))
_PART_SPEC = pl.BlockSpec((NC, R, D), lambda i: (0, i, 0))


def _wspec(a):
    return pl.BlockSpec(a.shape, lambda i: (0,) * a.ndim)


def _tc_self_parent(items, par_rows, ws, wp):
    def body(items_b, par_b,
             ws1, bs1, ws2, bs2, ws3, bs3,
             wp1, bp1, wp2, bp2, wp3, bp3,
             es_b, ep_b):
        es_b[...] = _mlp(items_b[...], ws1[...], bs1[...], ws2[...],
                         bs2[...], ws3[...], bs3[...])
        ep_b[...] = _mlp(par_b[...], wp1[...], bp1[...], wp2[...],
                         bp2[...], wp3[...], bp3[...])

    return pl.pallas_call(
        body,
        grid=(N // R,),
        in_specs=[_ROW_SPEC, _ROW_SPEC] + [_wspec(a) for a in ws + wp],
        out_specs=(_ROW_SPEC, _ROW_SPEC),
        out_shape=(jax.ShapeDtypeStruct((N, D), jnp.float32),
                   jax.ShapeDtypeStruct((N, D), jnp.float32)),
        compiler_params=pltpu.CompilerParams(
            dimension_semantics=("arbitrary",)),
    )(items, par_rows, *ws, *wp)


def _tc_rest(es, ep, child_part, ops_part, wc, wo, wm):
    def body(es_b, ep_b, ch_b, op_b,
             wc1, bc1, wc2, bc2, wc3, bc3,
             wo1, bo1, wo2, bo2, wo3, bo3,
             wm1, bm1, wm2, bm2, wm3, bm3,
             out_b):
        ec = _mlp(ch_b[0] + ch_b[1], wc1[...], bc1[...], wc2[...], bc2[...],
                  wc3[...], bc3[...])
        eo = _mlp(op_b[0] + op_b[1], wo1[...], bo1[...], wo2[...], bo2[...],
                  wo3[...], bo3[...])
        x = jnp.concatenate([ep_b[...], ec, eo, es_b[...]], axis=1)
        out = _mlp(x, wm1[...], bm1[...], wm2[...], bm2[...], wm3[...],
                   bm3[...])
        rid = pl.program_id(0) * R + lax.broadcasted_iota(jnp.int32, (R, 1), 0)
        out_b[...] = jnp.where(rid == N - 1, 0.0, out)

    return pl.pallas_call(
        body,
        grid=(N // R,),
        in_specs=[_ROW_SPEC, _ROW_SPEC, _PART_SPEC, _PART_SPEC]
        + [_wspec(a) for a in wc + wo + wm],
        out_specs=_ROW_SPEC,
        out_shape=jax.ShapeDtypeStruct((N, D), jnp.float32),
        compiler_params=pltpu.CompilerParams(
            dimension_semantics=("arbitrary",)),
    )(es, ep, child_part, ops_part, *wc, *wo, *wm)


def kernel(items, parents, operations, item_edge_index, op_edge_index, params):
    eshape = (NW, NHALF, HALF, CH)
    isrc = item_edge_index[1].reshape(eshape).astype(jnp.int32)
    idst = item_edge_index[0].reshape(eshape).astype(jnp.int32)
    osrc = op_edge_index[1].reshape(eshape).astype(jnp.int32)
    odst = op_edge_index[0].reshape(eshape).astype(jnp.int32)
    par_pad = jnp.concatenate(
        [parents.astype(jnp.int32),
         jnp.zeros((PAR_PAD - N,), jnp.int32)])
    zrows = jnp.zeros((STRIPE + TAIL, D), jnp.float32)

    def wl(name):
        p = params[name]
        out = []
        for l in ("l1", "l2", "l3"):
            w, b = p[l]
            out.append(w)
            out.append(b.reshape(1, -1))
        return out

    par_rows = _sc_parent_gather(items, par_pad)
    child_part, ops_part = _sc_sparse(
        items, operations, isrc, idst, osrc, odst, par_pad, zrows)

    es, ep = _tc_self_parent(items, par_rows, wl("mlp_self"),
                             wl("mlp_parent"))
    return _tc_rest(es, ep, child_part, ops_part, wl("mlp_children"),
                    wl("mlp_operations"), wl("mlp_combined"))


# speedup vs baseline: 1.1700x; 1.0019x over previous
import functools

import jax
import jax.numpy as jnp
from jax import lax
from jax.experimental import pallas as pl
from jax.experimental.pallas import tpu as pltpu
from jax.experimental.pallas import tpu_sc as plsc

N = 10000
D = 128
E = 320000
NC = 2
NS = 16
NW = NC * NS
EW = E // NW
CH = 100
NCHUNK = EW // CH
NHALF = 4
HALF = NCHUNK // NHALF
NBUF = 3
STRIPE = 624
TAIL = N - NS * STRIPE
PAR_PAD = 10240
PW = PAR_PAD // NW
PCH = 64
PSTEPS = PW // PCH


def _sc_parent_gather(items, parents):
    mesh = plsc.VectorSubcoreMesh(
        core_axis_name="c", subcore_axis_name="s",
        num_cores=NC, num_subcores=NS)

    @functools.partial(
        pl.kernel,
        out_type=jax.ShapeDtypeStruct((PAR_PAD, D), jnp.float32),
        mesh=mesh,
        scratch_types=[
            pltpu.VMEM((PW,), jnp.int32),
            pltpu.VMEM((PCH, D), jnp.float32),
            pltpu.SemaphoreType.DMA,
        ],
    )
    def pk(items_h, par_h, par_o, pidx, prow, sem):
        c = lax.axis_index("c")
        s = lax.axis_index("s")
        wid = s * NC + c
        pltpu.sync_copy(par_h.at[pl.ds(wid * PW, PW)], pidx)
        for t in range(PSTEPS):
            pltpu.async_copy(items_h.at[pidx.at[pl.ds(t * PCH, PCH)]],
                             prow, sem).wait()
            pltpu.sync_copy(prow,
                            par_o.at[pl.ds(wid * PW + t * PCH, PCH)])

    return pk(items, parents)


def _sc_sparse(items, operations, isrc, idst, osrc, odst, zrows):
    mesh = plsc.VectorSubcoreMesh(
        core_axis_name="c", subcore_axis_name="s",
        num_cores=NC, num_subcores=NS)

    @functools.partial(
        pl.kernel,
        out_type=(
            jax.ShapeDtypeStruct((NC, N, D), jnp.float32),
            jax.ShapeDtypeStruct((NC, N, D), jnp.float32),
        ),
        mesh=mesh,
        scratch_types=[
            pltpu.MemorySpace.VMEM_SHARED((N, D), jnp.float32),
            pltpu.VMEM((HALF, CH), jnp.int32),
            pltpu.VMEM((HALF, CH), jnp.int32),
            pltpu.VMEM((CH, D), jnp.float32),
            pltpu.VMEM((CH, D), jnp.float32),
            pltpu.VMEM((CH, D), jnp.float32),
            pltpu.SemaphoreType.DMA,
            pltpu.SemaphoreType.DMA,
            pltpu.SemaphoreType.DMA,
        ],
    )
    def k(items_h, ops_h, isrc_h, idst_h, osrc_h, odst_h, zrows_h,
          child_o, opsagg_o, acc, sidx, didx, rows, rows1, rows2,
          sem, sem1, sem2):
        c = lax.axis_index("c")
        s = lax.axis_index("s")
        wid = s * NC + c

        def stripe_copy(src_fn, dst_fn):
            pltpu.sync_copy(src_fn(0, STRIPE), dst_fn(0, STRIPE))

            @pl.when(s == NS - 1)
            def _():
                pltpu.sync_copy(src_fn(STRIPE, TAIL), dst_fn(STRIPE, TAIL))

        base = s * STRIPE

        def zero_stripe():
            stripe_copy(lambda o, n: zrows_h.at[pl.ds(o, n)],
                        lambda o, n: acc.at[pl.ds(base + o, n)])

        bufs = (rows, rows1, rows2)
        sems = (sem, sem1, sem2)

        def segsum(src_h, dst_h, table_h):
            def wait_gather(j, b):
                pltpu.make_async_copy(table_h.at[sidx.at[j]], bufs[b],
                                      sems[b]).wait()

            for g in range(NHALF):
                pltpu.sync_copy(src_h.at[wid, g], sidx)
                pltpu.sync_copy(dst_h.at[wid, g], didx)

                for j in range(NBUF):
                    pltpu.async_copy(table_h.at[sidx.at[j]], bufs[j],
                                     sems[j])

                def triple(i, _):
                    j0 = i * NBUF
                    for b in range(NBUF):
                        j = j0 + b
                        wait_gather(j, b)
                        pltpu.sync_copy(bufs[b], acc.at[didx.at[j]],
                                        add=True)

                        @pl.when(j + NBUF < HALF)
                        def _(jn=j + NBUF, bb=b):
                            pltpu.async_copy(table_h.at[sidx.at[jn]],
                                             bufs[bb], sems[bb])
                    return ()

                lax.fori_loop(0, HALF // NBUF, triple, ())
                jt = HALF - 1
                bt = jt % NBUF
                wait_gather(jt, bt)
                pltpu.sync_copy(bufs[bt], acc.at[didx.at[jt]], add=True)

        zero_stripe()
        plsc.subcore_barrier()
        segsum(isrc_h, idst_h, items_h)
        plsc.subcore_barrier()
        stripe_copy(lambda o, n: acc.at[pl.ds(base + o, n)],
                    lambda o, n: child_o.at[c, pl.ds(base + o, n)])
        zero_stripe()
        plsc.subcore_barrier()

        segsum(osrc_h, odst_h, ops_h)
        plsc.subcore_barrier()
        stripe_copy(lambda o, n: acc.at[pl.ds(base + o, n)],
                    lambda o, n: opsagg_o.at[c, pl.ds(base + o, n)])

    return k(items, operations, isrc, idst, osrc, odst, zrows)


def _elu(x):
    return jnp.where(x > 0, x, jnp.exp(x) - 1.0)


def _mlp(x, w1, b1, w2, b2, w3, b3):
    x = _elu(jnp.dot(x, w1, preferred_element_type=jnp.float32) + b1)
    x = _elu(jnp.dot(x, w2, preferred_element_type=jnp.float32) + b2)
    return jnp.dot(x, w3, preferred_element_type=jnp.float32) + b3


R = 2000


_ROW_SPEC = pl.BlockSpec((R, D), lambda i: (i, 0))
_PART_SPEC = pl.BlockSpec((NC, R, D), lambda i: (0, i, 0))


def _wspec(a):
    return pl.BlockSpec(a.shape, lambda i: (0,) * a.ndim)


def _tc_self_parent(items, par_rows, ws, wp):
    def body(items_b, par_b,
             ws1, bs1, ws2, bs2, ws3, bs3,
             wp1, bp1, wp2, bp2, wp3, bp3,
             es_b, ep_b):
        es_b[...] = _mlp(items_b[...], ws1[...], bs1[...], ws2[...],
                         bs2[...], ws3[...], bs3[...])
        ep_b[...] = _mlp(par_b[...], wp1[...], bp1[...], wp2[...],
                         bp2[...], wp3[...], bp3[...])

    return pl.pallas_call(
        body,
        grid=(N // R,),
        in_specs=[_ROW_SPEC, _ROW_SPEC] + [_wspec(a) for a in ws + wp],
        out_specs=(_ROW_SPEC, _ROW_SPEC),
        out_shape=(jax.ShapeDtypeStruct((N, D), jnp.float32),
                   jax.ShapeDtypeStruct((N, D), jnp.float32)),
        compiler_params=pltpu.CompilerParams(
            dimension_semantics=("arbitrary",)),
    )(items, par_rows, *ws, *wp)


def _tc_rest(es, ep, child_part, ops_part, wc, wo, wm):
    def body(es_b, ep_b, ch_b, op_b,
             wc1, bc1, wc2, bc2, wc3, bc3,
             wo1, bo1, wo2, bo2, wo3, bo3,
             wm1, bm1, wm2, bm2, wm3, bm3,
             out_b):
        ec = _mlp(ch_b[0] + ch_b[1], wc1[...], bc1[...], wc2[...], bc2[...],
                  wc3[...], bc3[...])
        eo = _mlp(op_b[0] + op_b[1], wo1[...], bo1[...], wo2[...], bo2[...],
                  wo3[...], bo3[...])
        x = jnp.concatenate([ep_b[...], ec, eo, es_b[...]], axis=1)
        out = _mlp(x, wm1[...], bm1[...], wm2[...], bm2[...], wm3[...],
                   bm3[...])
        rid = pl.program_id(0) * R + lax.broadcasted_iota(jnp.int32, (R, 1), 0)
        out_b[...] = jnp.where(rid == N - 1, 0.0, out)

    return pl.pallas_call(
        body,
        grid=(N // R,),
        in_specs=[_ROW_SPEC, _ROW_SPEC, _PART_SPEC, _PART_SPEC]
        + [_wspec(a) for a in wc + wo + wm],
        out_specs=_ROW_SPEC,
        out_shape=jax.ShapeDtypeStruct((N, D), jnp.float32),
        compiler_params=pltpu.CompilerParams(
            dimension_semantics=("arbitrary",)),
    )(es, ep, child_part, ops_part, *wc, *wo, *wm)


def kernel(items, parents, operations, item_edge_index, op_edge_index, params):
    eshape = (NW, NHALF, HALF, CH)
    isrc = item_edge_index[1].reshape(eshape).astype(jnp.int32)
    idst = item_edge_index[0].reshape(eshape).astype(jnp.int32)
    osrc = op_edge_index[1].reshape(eshape).astype(jnp.int32)
    odst = op_edge_index[0].reshape(eshape).astype(jnp.int32)
    par_pad = jnp.concatenate(
        [parents.astype(jnp.int32),
         jnp.zeros((PAR_PAD - N,), jnp.int32)])
    zrows = jnp.zeros((STRIPE + TAIL, D), jnp.float32)

    def wl(name):
        p = params[name]
        out = []
        for l in ("l1", "l2", "l3"):
            w, b = p[l]
            out.append(w)
            out.append(b.reshape(1, -1))
        return out

    par_rows = _sc_parent_gather(items, par_pad)
    child_part, ops_part = _sc_sparse(
        items, operations, isrc, idst, osrc, odst, zrows)

    es, ep = _tc_self_parent(items, par_rows, wl("mlp_self"),
                             wl("mlp_parent"))
    return _tc_rest(es, ep, child_part, ops_part, wl("mlp_children"),
                    wl("mlp_operations"), wl("mlp_combined"))
